# Initial kernel scaffold; baseline (speedup 1.0000x reference)
import functools

import jax
import jax.numpy as jnp
from jax import lax
from jax.experimental import pallas as pl
from jax.experimental.pallas import tpu as pltpu
from jax.experimental.pallas import tpu_sc as plsc

B = 4 * 8192
D = 1024
NC, NS = 2, 16
NW = NC * NS
B_PER_W = B // NW
CHUNK = 32
NCHUNK = B_PER_W // CHUNK


def _gather_kernel(table_hbm, idx_hbm, out_hbm, idx_v, buf0, buf1, sem0, sem1):
    wid = lax.axis_index("s") * NC + lax.axis_index("c")
    base = wid * B_PER_W
    pltpu.sync_copy(idx_hbm.at[pl.ds(base, B_PER_W)], idx_v)

    pltpu.async_copy(table_hbm.at[idx_v.at[pl.ds(0, CHUNK)]], buf0, sem0).start()

    @pl.loop(0, NCHUNK, step=2)
    def _(g):
        pltpu.async_copy(
            table_hbm.at[idx_v.at[pl.ds((g + 1) * CHUNK, CHUNK)]], buf1, sem1
        ).start()
        pltpu.make_async_copy(
            table_hbm.at[idx_v.at[pl.ds(g * CHUNK, CHUNK)]], buf0, sem0
        ).wait()
        pltpu.sync_copy(buf0, out_hbm.at[pl.ds(base + g * CHUNK, CHUNK)])

        @pl.when(g + 2 < NCHUNK)
        def _():
            pltpu.async_copy(
                table_hbm.at[idx_v.at[pl.ds((g + 2) * CHUNK, CHUNK)]], buf0, sem0
            ).start()

        pltpu.make_async_copy(
            table_hbm.at[idx_v.at[pl.ds((g + 1) * CHUNK, CHUNK)]], buf1, sem1
        ).wait()
        pltpu.sync_copy(buf1, out_hbm.at[pl.ds(base + (g + 1) * CHUNK, CHUNK)])


def kernel(position_ids, embedding_weight):
    idx = position_ids.reshape(B).astype(jnp.int32)
    mesh = plsc.VectorSubcoreMesh(core_axis_name="c", subcore_axis_name="s")
    k = functools.partial(
        pl.kernel,
        mesh=mesh,
        out_type=jax.ShapeDtypeStruct((B, D), jnp.float32),
        scratch_types=[
            pltpu.VMEM((B_PER_W,), jnp.int32),
            pltpu.VMEM((CHUNK, D), jnp.float32),
            pltpu.VMEM((CHUNK, D), jnp.float32),
            pltpu.SemaphoreType.DMA,
            pltpu.SemaphoreType.DMA,
        ],
    )(_gather_kernel)
    out = k(embedding_weight, idx)
    return out.reshape(4, 8192, D)


# speedup vs baseline: 2.3850x; 2.3850x over previous
import functools

import jax
import jax.numpy as jnp
from jax import lax
from jax.experimental import pallas as pl
from jax.experimental.pallas import tpu as pltpu
from jax.experimental.pallas import tpu_sc as plsc

B = 4 * 8192
D = 1024
NC, NS = 2, 16
NW = NC * NS
B_PER_W = B // NW
CHUNK = 32
NCHUNK = B_PER_W // CHUNK


def _gather_kernel(table_hbm, idx_hbm, out_hbm, idx_v, buf0, buf1, sem0, sem1):
    wid = lax.axis_index("s") * NC + lax.axis_index("c")
    base = wid * B_PER_W
    pltpu.sync_copy(idx_hbm.at[pl.ds(base, B_PER_W)], idx_v)

    def gather_start(g, buf, sem):
        pltpu.make_async_copy(
            table_hbm.at[idx_v.at[pl.ds(g * CHUNK, CHUNK)]], buf, sem
        ).start()

    def gather_wait(g, buf, sem):
        pltpu.make_async_copy(
            table_hbm.at[idx_v.at[pl.ds(g * CHUNK, CHUNK)]], buf, sem
        ).wait()

    gather_start(0, buf0, sem0)

    @pl.loop(0, NCHUNK, step=2)
    def _(g):
        gather_start(g + 1, buf1, sem1)
        gather_wait(g, buf0, sem0)
        pltpu.sync_copy(buf0, out_hbm.at[pl.ds(base + g * CHUNK, CHUNK)])

        @pl.when(g + 2 < NCHUNK)
        def _():
            gather_start(g + 2, buf0, sem0)

        gather_wait(g + 1, buf1, sem1)
        pltpu.sync_copy(buf1, out_hbm.at[pl.ds(base + (g + 1) * CHUNK, CHUNK)])


def kernel(position_ids, embedding_weight):
    idx = position_ids.reshape(B).astype(jnp.int32)
    mesh = plsc.VectorSubcoreMesh(core_axis_name="c", subcore_axis_name="s")
    k = functools.partial(
        pl.kernel,
        mesh=mesh,
        out_type=jax.ShapeDtypeStruct((B, D), jnp.float32),
        scratch_types=[
            pltpu.VMEM((B_PER_W,), jnp.int32),
            pltpu.VMEM((CHUNK, D), jnp.float32),
            pltpu.VMEM((CHUNK, D), jnp.float32),
            pltpu.SemaphoreType.DMA,
            pltpu.SemaphoreType.DMA,
        ],
    )(_gather_kernel)
    out = k(embedding_weight, idx)
    return out.reshape(4, 8192, D)


# speedup vs baseline: 2.3880x; 1.0013x over previous
import functools

import jax
import jax.numpy as jnp
from jax import lax
from jax.experimental import pallas as pl
from jax.experimental.pallas import tpu as pltpu
from jax.experimental.pallas import tpu_sc as plsc

B = 4 * 8192
D = 1024
NC, NS = 2, 16
NW = NC * NS
B_PER_W = B // NW
CHUNK = 16
NCHUNK = B_PER_W // CHUNK
NBUF = 4


def _gather_kernel(table_hbm, idx_hbm, out_hbm, idx_v, *rest):
    bufs = rest[:NBUF]
    gsems = rest[NBUF:2 * NBUF]
    wsems = rest[2 * NBUF:3 * NBUF]

    wid = lax.axis_index("s") * NC + lax.axis_index("c")
    base = wid * B_PER_W
    pltpu.sync_copy(idx_hbm.at[pl.ds(base, B_PER_W)], idx_v)

    def gather_cp(c, j):
        return pltpu.make_async_copy(
            table_hbm.at[idx_v.at[pl.ds(c * CHUNK, CHUNK)]], bufs[j], gsems[j]
        )

    def write_cp(c, j):
        return pltpu.make_async_copy(
            bufs[j], out_hbm.at[pl.ds(base + c * CHUNK, CHUNK)], wsems[j]
        )

    for j in range(NBUF - 1):
        gather_cp(j, j).start()

    @pl.loop(0, NCHUNK, step=NBUF)
    def _(g):
        for j in range(NBUF):
            cc = g + j
            gather_cp(cc, j).wait()
            write_cp(cc, j).start()
            jn = (j + NBUF - 1) % NBUF
            @pl.when(cc + NBUF - 1 < NCHUNK)
            def _():
                @pl.when(cc >= 1)
                def _():
                    write_cp(cc - 1, jn).wait()
                gather_cp(cc + NBUF - 1, jn).start()

    for c in range(NCHUNK - NBUF, NCHUNK):
        write_cp(c, c % NBUF).wait()


def kernel(position_ids, embedding_weight):
    idx = position_ids.reshape(B).astype(jnp.int32)
    mesh = plsc.VectorSubcoreMesh(core_axis_name="c", subcore_axis_name="s")
    k = functools.partial(
        pl.kernel,
        mesh=mesh,
        out_type=jax.ShapeDtypeStruct((B, D), jnp.float32),
        scratch_types=(
            [pltpu.VMEM((B_PER_W,), jnp.int32)]
            + [pltpu.VMEM((CHUNK, D), jnp.float32) for _ in range(NBUF)]
            + [pltpu.SemaphoreType.DMA for _ in range(2 * NBUF)]
        ),
    )(_gather_kernel)
    out = k(embedding_weight, idx)
    return out.reshape(4, 8192, D)
